# baseline (speedup 1.0000x reference)
import functools

import jax
import jax.numpy as jnp
from jax import lax
from jax.experimental import pallas as pl
from jax.experimental.pallas import tpu as pltpu
from jax.experimental.pallas import tpu_sc as plsc

NC = 1
NS = 16
CW = 64
SG = 8
NB = 4
ZR = 8
D2 = 128


def _edge_rows(e):
    return -(-e // (NC * NS * CW * SG)) * SG


@functools.lru_cache(maxsize=None)
def _make_sc_segsum(npad, rpt, nhalves, with_deg=False):
    nsg = rpt // SG
    npt = npad // NS
    nzc = npt // ZR

    mesh = plsc.VectorSubcoreMesh(core_axis_name="c", subcore_axis_name="s", num_cores=NC)

    scratch = (
        [pltpu.VMEM((SG, CW), jnp.int32) for _ in range(4)]
        + [pltpu.VMEM((CW, D2), jnp.float32) for _ in range(NB)]
        + [pltpu.VMEM((ZR, D2), jnp.float32)]
        + [pltpu.SemaphoreType.DMA for _ in range(4 + 2 * NB)]
        + [pltpu.VMEM_SHARED((npad, D2), jnp.float32)]
    )

    def body(*refs):
        tables = refs[:nhalves]
        srcm, dstm = refs[nhalves:nhalves + 2]
        out_hbm = refs[nhalves + 2]
        k = nhalves + 3
        if with_deg:
            deg_hbm = refs[k]
            k += 1
        src_s = refs[k:k + 2]
        dst_s = refs[k + 2:k + 4]
        dbuf = refs[k + 4:k + 4 + NB]
        zbuf = refs[k + 4 + NB]
        isem = refs[k + 5 + NB:k + 7 + NB]
        jsem = refs[k + 7 + NB:k + 9 + NB]
        dsem = refs[k + 9 + NB:k + 9 + 2 * NB]
        ssem = refs[k + 9 + 2 * NB:k + 9 + 3 * NB]
        acc = refs[k + 9 + 3 * NB]

        c = lax.axis_index("c")
        s = lax.axis_index("s")
        row0 = (c * NS + s) * rpt

        @pl.loop(0, ZR)
        def _z(r):
            for q in range(D2 // 16):
                zbuf[r, pl.ds(q * 16, 16)] = jnp.zeros((16,), jnp.float32)

        r0 = s * npt

        def zero_acc():
            for q in range(nzc):
                pltpu.sync_copy(zbuf, acc.at[pl.ds(r0 + q * ZR, ZR)])

        zero_acc()
        plsc.subcore_barrier()

        for h in range(nhalves):
            if h > 0:
                zero_acc()
                plsc.subcore_barrier()
            table = tables[h]

            for t in range(2):
                pltpu.async_copy(srcm.at[pl.ds(row0 + t * SG, SG)],
                                 src_s[t], isem[t])
                pltpu.async_copy(dstm.at[pl.ds(row0 + t * SG, SG)],
                                 dst_s[t], jsem[t])
            pltpu.make_async_copy(srcm.at[pl.ds(row0, SG)], src_s[0],
                                  isem[0]).wait()
            pltpu.make_async_copy(dstm.at[pl.ds(row0, SG)], dst_s[0],
                                  jsem[0]).wait()
            for b in range(2):
                pltpu.async_copy(table.at[src_s[0].at[b]], dbuf[b], dsem[b])

            @pl.loop(0, (nsg + 1) // 2)
            def _sg2(gp):
                for par in range(2):
                    sg = gp * 2 + par
                    run = (sg < nsg) if nsg % 2 else None

                    def do_sg(sg=sg, par=par):
                        nxt = 1 - par
                        for r in range(SG):
                            slot = r % NB
                            gslot = (r + 2) % NB
                            if r == 6:
                                @pl.when(sg + 1 < nsg)
                                def _():
                                    pltpu.make_async_copy(
                                        srcm.at[pl.ds(row0 + (sg + 1) * SG,
                                                      SG)],
                                        src_s[nxt], isem[nxt]).wait()
                                    pltpu.make_async_copy(
                                        dstm.at[pl.ds(row0 + (sg + 1) * SG,
                                                      SG)],
                                        dst_s[nxt], jsem[nxt]).wait()

                            def drain_and_gather(idx_ref, row, sg=sg, r=r,
                                                 gslot=gslot):
                                pltpu.make_async_copy(
                                    dbuf[gslot], acc.at[dst_s[par].at[0]],
                                    ssem[gslot]).wait()
                                pltpu.async_copy(table.at[idx_ref.at[row]],
                                                 dbuf[gslot], dsem[gslot])

                            def gather_only(idx_ref, row, gslot=gslot):
                                pltpu.async_copy(table.at[idx_ref.at[row]],
                                                 dbuf[gslot], dsem[gslot])

                            if r < SG - 2:
                                if r < 2:
                                    @pl.when(sg > 0)
                                    def _(r=r, gslot=gslot):
                                        drain_and_gather(src_s[par], r + 2)

                                    @pl.when(sg == 0)
                                    def _(r=r, gslot=gslot):
                                        gather_only(src_s[par], r + 2)
                                else:
                                    drain_and_gather(src_s[par], r + 2)
                            else:
                                @pl.when(sg + 1 < nsg)
                                def _(r=r, gslot=gslot):
                                    drain_and_gather(src_s[nxt], r - 6)

                            pltpu.make_async_copy(table.at[src_s[par].at[r]],
                                                  dbuf[slot],
                                                  dsem[slot]).wait()
                            pltpu.async_copy(dbuf[slot],
                                             acc.at[dst_s[par].at[r]],
                                             ssem[slot], add=True)

                        @pl.when(sg + 2 < nsg)
                        def _():
                            pltpu.async_copy(
                                srcm.at[pl.ds(row0 + (sg + 2) * SG, SG)],
                                src_s[par], isem[par])
                            pltpu.async_copy(
                                dstm.at[pl.ds(row0 + (sg + 2) * SG, SG)],
                                dst_s[par], jsem[par])

                    if run is None:
                        do_sg()
                    else:
                        pl.when(run)(do_sg)

            for b in range(NB):
                pltpu.make_async_copy(dbuf[b], acc.at[dst_s[0].at[0]],
                                      ssem[b]).wait()

            plsc.subcore_barrier()
            pltpu.sync_copy(acc.at[pl.ds(r0, npt)],
                            out_hbm.at[h * NC + c, pl.ds(r0, npt)])

        if with_deg:
            zero_acc()

            @pl.loop(0, CW)
            def _fo(r):
                for q in range(D2 // 16):
                    dbuf[0][r, pl.ds(q * 16, 16)] = jnp.ones((16,),
                                                             jnp.float32)

            plsc.subcore_barrier()

            for t in range(2):
                if t < nsg:
                    pltpu.async_copy(dstm.at[pl.ds(row0 + t * SG, SG)],
                                     dst_s[t], jsem[t])

            @pl.loop(0, (nsg + 1) // 2)
            def _sgd(gp):
                for par in range(2):
                    sg = gp * 2 + par
                    run = (sg < nsg) if nsg % 2 else None

                    def do_sg(sg=sg, par=par):
                        pltpu.make_async_copy(
                            dstm.at[pl.ds(row0 + sg * SG, SG)],
                            dst_s[par], jsem[par]).wait()
                        for q in range(SG):
                            pltpu.sync_copy(dbuf[0],
                                            acc.at[dst_s[par].at[q]],
                                            add=True)

                        @pl.when(sg + 2 < nsg)
                        def _():
                            pltpu.async_copy(
                                dstm.at[pl.ds(row0 + (sg + 2) * SG, SG)],
                                dst_s[par], jsem[par])

                    if run is None:
                        do_sg()
                    else:
                        pl.when(run)(do_sg)

            plsc.subcore_barrier()
            pltpu.sync_copy(acc.at[pl.ds(r0, npt)],
                            deg_hbm.at[c, pl.ds(r0, npt)])

    out_type = [jax.ShapeDtypeStruct((NC * nhalves, npad, D2), jnp.float32)]
    if with_deg:
        out_type.append(jax.ShapeDtypeStruct((NC, npad, D2), jnp.float32))
    return pl.kernel(body,
                     out_type=tuple(out_type) if with_deg else out_type[0],
                     mesh=mesh, scratch_types=scratch)


def _inv_deg(deg_ref):
    d = deg_ref[0][:, 0:1]
    for c in range(1, NC):
        d = d + deg_ref[c][:, 0:1]
    return 1.0 / jnp.maximum(d, 1.0)


def _psum(a_ref, h):
    p = a_ref[h * NC]
    for c in range(1, NC):
        p = p + a_ref[h * NC + c]
    return p


def _k2_body(a_ref, deg_ref, x_ref, wl_ref, wr_ref, b_ref, lo_ref, hi_ref):
    inv = _inv_deg(deg_ref)
    m = jnp.dot(_psum(a_ref, 0), wl_ref[...],
                preferred_element_type=jnp.float32)
    h = m * inv + jnp.dot(x_ref[...], wr_ref[...],
                          preferred_element_type=jnp.float32) + b_ref[...]
    h = jnp.maximum(h, 0.0)
    lo_ref[...] = h[:, :128]
    hi_ref[...] = h[:, 128:]


def _k4_body(a_ref, deg_ref, hlo_ref, hhi_ref, wl2_ref, wr2_ref, b2_ref,
             wl3_ref, wr3_ref, b3_ref, t3_ref, s3_ref):
    inv = _inv_deg(deg_ref)
    m = (jnp.dot(_psum(a_ref, 0), wl2_ref[0:128],
                 preferred_element_type=jnp.float32)
         + jnp.dot(_psum(a_ref, 1), wl2_ref[128:256],
                   preferred_element_type=jnp.float32))
    r = (jnp.dot(hlo_ref[...], wr2_ref[0:128],
                 preferred_element_type=jnp.float32)
         + jnp.dot(hhi_ref[...], wr2_ref[128:256],
                   preferred_element_type=jnp.float32))
    h2 = jnp.maximum(m * inv + r + b2_ref[...], 0.0)
    t3_ref[...] = jnp.dot(h2, wl3_ref[...], preferred_element_type=jnp.float32)
    s3_ref[...] = jnp.dot(h2, wr3_ref[...],
                          preferred_element_type=jnp.float32) + b3_ref[...]


def _k6_body(a_ref, deg_ref, s3_ref, out_ref):
    inv = _inv_deg(deg_ref)
    out_ref[...] = _psum(a_ref, 0) * inv + s3_ref[...]


@functools.lru_cache(maxsize=None)
def _make_k2(n, npad, bn=400):
    return pl.pallas_call(
        _k2_body,
        grid=(n // bn,),
        in_specs=[
            pl.BlockSpec((NC, bn, 128), lambda i: (0, i, 0)),
            pl.BlockSpec((NC, bn, 128), lambda i: (0, i, 0)),
            pl.BlockSpec((bn, 128), lambda i: (i, 0)),
            pl.BlockSpec((128, 256), lambda i: (0, 0)),
            pl.BlockSpec((128, 256), lambda i: (0, 0)),
            pl.BlockSpec((1, 256), lambda i: (0, 0)),
        ],
        out_specs=[
            pl.BlockSpec((bn, 128), lambda i: (i, 0)),
            pl.BlockSpec((bn, 128), lambda i: (i, 0)),
        ],
        out_shape=[
            jax.ShapeDtypeStruct((n, 128), jnp.float32),
            jax.ShapeDtypeStruct((n, 128), jnp.float32),
        ],
    )


@functools.lru_cache(maxsize=None)
def _make_k4(n, npad, bn=400):
    return pl.pallas_call(
        _k4_body,
        grid=(n // bn,),
        in_specs=[
            pl.BlockSpec((2 * NC, bn, 128), lambda i: (0, i, 0)),
            pl.BlockSpec((NC, bn, 128), lambda i: (0, i, 0)),
            pl.BlockSpec((bn, 128), lambda i: (i, 0)),
            pl.BlockSpec((bn, 128), lambda i: (i, 0)),
            pl.BlockSpec((256, 256), lambda i: (0, 0)),
            pl.BlockSpec((256, 256), lambda i: (0, 0)),
            pl.BlockSpec((1, 256), lambda i: (0, 0)),
            pl.BlockSpec((256, 128), lambda i: (0, 0)),
            pl.BlockSpec((256, 128), lambda i: (0, 0)),
            pl.BlockSpec((1, 128), lambda i: (0, 0)),
        ],
        out_specs=[
            pl.BlockSpec((bn, 128), lambda i: (i, 0)),
            pl.BlockSpec((bn, 128), lambda i: (i, 0)),
        ],
        out_shape=[
            jax.ShapeDtypeStruct((n, 128), jnp.float32),
            jax.ShapeDtypeStruct((n, 128), jnp.float32),
        ],
    )


@functools.lru_cache(maxsize=None)
def _make_k6(n, npad, bn=400):
    return pl.pallas_call(
        _k6_body,
        grid=(n // bn,),
        in_specs=[
            pl.BlockSpec((NC, bn, 128), lambda i: (0, i, 0)),
            pl.BlockSpec((NC, bn, 128), lambda i: (0, i, 0)),
            pl.BlockSpec((bn, 128), lambda i: (i, 0)),
        ],
        out_specs=pl.BlockSpec((bn, 128), lambda i: (i, 0)),
        out_shape=jax.ShapeDtypeStruct((n, 128), jnp.float32),
    )


@jax.jit
def kernel(x, edge_index, W_l1, W_r1, b1, W_l2, W_r2, b2, W_l3, W_r3, b3):
    n, d_in = x.shape
    e = edge_index.shape[1]

    npad = -(-n // 128) * 128

    rpt = _edge_rows(e)
    epad = NC * NS * CW * rpt
    src = jnp.concatenate(
        [edge_index[0], jnp.zeros((epad - e,), jnp.int32)])
    dst = jnp.concatenate(
        [edge_index[1], jnp.full((epad - e,), npad - 1, jnp.int32)])
    srcm = src.reshape(epad // CW, CW)
    dstm = dst.reshape(epad // CW, CW)

    agg1, deg = _make_sc_segsum(npad, rpt, 1, True)(x, srcm, dstm)
    h1lo, h1hi = _make_k2(n, npad)(agg1, deg, x, W_l1, W_r1, b1.reshape(1, -1))
    agg2 = _make_sc_segsum(npad, rpt, 2)(h1lo, h1hi, srcm, dstm)
    t3, s3 = _make_k4(n, npad)(agg2, deg, h1lo, h1hi, W_l2, W_r2,
                               b2.reshape(1, -1), W_l3, W_r3,
                               b3.reshape(1, -1))
    agg3 = _make_sc_segsum(npad, rpt, 1)(t3, srcm, dstm)
    return _make_k6(n, npad)(agg3, deg, s3)


# speedup vs baseline: 1.1378x; 1.1378x over previous
import functools

import jax
import jax.numpy as jnp
from jax import lax
from jax.experimental import pallas as pl
from jax.experimental.pallas import tpu as pltpu
from jax.experimental.pallas import tpu_sc as plsc

NC = 2
NS = 16
CW = 64
SG = 8
NB = 4
ZR = 8
D2 = 128


def _edge_rows(e):
    return -(-e // (NC * NS * CW * SG)) * SG


@functools.lru_cache(maxsize=None)
def _make_sc_segsum(npad, rpt0, rpt1, nhalves, with_deg=False):
    npt = npad // NS
    nzc = npt // ZR

    mesh = plsc.VectorSubcoreMesh(core_axis_name="c", subcore_axis_name="s", num_cores=NC)

    scratch = (
        [pltpu.VMEM((SG, CW), jnp.int32) for _ in range(4)]
        + [pltpu.VMEM((CW, D2), jnp.float32) for _ in range(NB)]
        + [pltpu.VMEM((ZR, D2), jnp.float32)]
        + [pltpu.SemaphoreType.DMA for _ in range(4 + 2 * NB)]
        + [pltpu.VMEM_SHARED((npad, D2), jnp.float32)]
    )

    def body(*refs):
        tables = refs[:nhalves]
        srcm, dstm = refs[nhalves:nhalves + 2]
        out_hbm = refs[nhalves + 2]
        k = nhalves + 3
        if with_deg:
            deg_hbm = refs[k]
            k += 1
        src_s = refs[k:k + 2]
        dst_s = refs[k + 2:k + 4]
        dbuf = refs[k + 4:k + 4 + NB]
        zbuf = refs[k + 4 + NB]
        isem = refs[k + 5 + NB:k + 7 + NB]
        jsem = refs[k + 7 + NB:k + 9 + NB]
        dsem = refs[k + 9 + NB:k + 9 + 2 * NB]
        ssem = refs[k + 9 + 2 * NB:k + 9 + 3 * NB]
        acc = refs[k + 9 + 3 * NB]

        c = lax.axis_index("c")
        s = lax.axis_index("s")
        nsg = jnp.where(c == 0, rpt0 // SG, rpt1 // SG)
        row0 = jnp.where(c == 0, s * rpt0, NS * rpt0 + s * rpt1)

        @pl.loop(0, ZR)
        def _z(r):
            for q in range(D2 // 16):
                zbuf[r, pl.ds(q * 16, 16)] = jnp.zeros((16,), jnp.float32)

        r0 = s * npt

        def zero_acc():
            for q in range(nzc):
                pltpu.sync_copy(zbuf, acc.at[pl.ds(r0 + q * ZR, ZR)])

        zero_acc()
        plsc.subcore_barrier()

        for h in range(nhalves):
            if h > 0:
                zero_acc()
                plsc.subcore_barrier()
            table = tables[h]

            for t in range(2):
                pltpu.async_copy(srcm.at[pl.ds(row0 + t * SG, SG)],
                                 src_s[t], isem[t])
                pltpu.async_copy(dstm.at[pl.ds(row0 + t * SG, SG)],
                                 dst_s[t], jsem[t])
            pltpu.make_async_copy(srcm.at[pl.ds(row0, SG)], src_s[0],
                                  isem[0]).wait()
            pltpu.make_async_copy(dstm.at[pl.ds(row0, SG)], dst_s[0],
                                  jsem[0]).wait()
            for b in range(2):
                pltpu.async_copy(table.at[src_s[0].at[b]], dbuf[b], dsem[b])

            @pl.loop(0, (nsg + 1) // 2)
            def _sg2(gp):
                for par in range(2):
                    sg = gp * 2 + par
                    run = sg < nsg

                    def do_sg(sg=sg, par=par):
                        nxt = 1 - par
                        for r in range(SG):
                            slot = r % NB
                            gslot = (r + 2) % NB
                            if r == 6:
                                @pl.when(sg + 1 < nsg)
                                def _():
                                    pltpu.make_async_copy(
                                        srcm.at[pl.ds(row0 + (sg + 1) * SG,
                                                      SG)],
                                        src_s[nxt], isem[nxt]).wait()
                                    pltpu.make_async_copy(
                                        dstm.at[pl.ds(row0 + (sg + 1) * SG,
                                                      SG)],
                                        dst_s[nxt], jsem[nxt]).wait()

                            def drain_and_gather(idx_ref, row, sg=sg, r=r,
                                                 gslot=gslot):
                                pltpu.make_async_copy(
                                    dbuf[gslot], acc.at[dst_s[par].at[0]],
                                    ssem[gslot]).wait()
                                pltpu.async_copy(table.at[idx_ref.at[row]],
                                                 dbuf[gslot], dsem[gslot])

                            def gather_only(idx_ref, row, gslot=gslot):
                                pltpu.async_copy(table.at[idx_ref.at[row]],
                                                 dbuf[gslot], dsem[gslot])

                            if r < SG - 2:
                                if r < 2:
                                    @pl.when(sg > 0)
                                    def _(r=r, gslot=gslot):
                                        drain_and_gather(src_s[par], r + 2)

                                    @pl.when(sg == 0)
                                    def _(r=r, gslot=gslot):
                                        gather_only(src_s[par], r + 2)
                                else:
                                    drain_and_gather(src_s[par], r + 2)
                            else:
                                @pl.when(sg + 1 < nsg)
                                def _(r=r, gslot=gslot):
                                    drain_and_gather(src_s[nxt], r - 6)

                            pltpu.make_async_copy(table.at[src_s[par].at[r]],
                                                  dbuf[slot],
                                                  dsem[slot]).wait()
                            pltpu.async_copy(dbuf[slot],
                                             acc.at[dst_s[par].at[r]],
                                             ssem[slot], add=True)

                        @pl.when(sg + 2 < nsg)
                        def _():
                            pltpu.async_copy(
                                srcm.at[pl.ds(row0 + (sg + 2) * SG, SG)],
                                src_s[par], isem[par])
                            pltpu.async_copy(
                                dstm.at[pl.ds(row0 + (sg + 2) * SG, SG)],
                                dst_s[par], jsem[par])

                    pl.when(run)(do_sg)

            for b in range(NB):
                pltpu.make_async_copy(dbuf[b], acc.at[dst_s[0].at[0]],
                                      ssem[b]).wait()

            plsc.subcore_barrier()
            pltpu.sync_copy(acc.at[pl.ds(r0, npt)],
                            out_hbm.at[h * NC + c, pl.ds(r0, npt)])

        if with_deg:
            zero_acc()

            @pl.loop(0, CW)
            def _fo(r):
                for q in range(D2 // 16):
                    dbuf[0][r, pl.ds(q * 16, 16)] = jnp.ones((16,),
                                                             jnp.float32)

            plsc.subcore_barrier()

            for t in range(2):
                pltpu.async_copy(dstm.at[pl.ds(row0 + t * SG, SG)],
                                 dst_s[t], jsem[t])

            @pl.loop(0, (nsg + 1) // 2)
            def _sgd(gp):
                for par in range(2):
                    sg = gp * 2 + par
                    run = sg < nsg

                    def do_sg(sg=sg, par=par):
                        pltpu.make_async_copy(
                            dstm.at[pl.ds(row0 + sg * SG, SG)],
                            dst_s[par], jsem[par]).wait()
                        for q in range(SG):
                            pltpu.sync_copy(dbuf[0],
                                            acc.at[dst_s[par].at[q]],
                                            add=True)

                        @pl.when(sg + 2 < nsg)
                        def _():
                            pltpu.async_copy(
                                dstm.at[pl.ds(row0 + (sg + 2) * SG, SG)],
                                dst_s[par], jsem[par])

                    pl.when(run)(do_sg)

            plsc.subcore_barrier()
            pltpu.sync_copy(acc.at[pl.ds(r0, npt)],
                            deg_hbm.at[c, pl.ds(r0, npt)])

    out_type = [jax.ShapeDtypeStruct((NC * nhalves, npad, D2), jnp.float32)]
    if with_deg:
        out_type.append(jax.ShapeDtypeStruct((NC, npad, D2), jnp.float32))
    return pl.kernel(body,
                     out_type=tuple(out_type) if with_deg else out_type[0],
                     mesh=mesh, scratch_types=scratch)


def _inv_deg(deg_ref):
    d = deg_ref[0][:, 0:1]
    for c in range(1, NC):
        d = d + deg_ref[c][:, 0:1]
    return 1.0 / jnp.maximum(d, 1.0)


def _psum(a_ref, h):
    p = a_ref[h * NC]
    for c in range(1, NC):
        p = p + a_ref[h * NC + c]
    return p


def _k2_body(a_ref, deg_ref, x_ref, wl_ref, wr_ref, b_ref, lo_ref, hi_ref):
    inv = _inv_deg(deg_ref)
    m = jnp.dot(_psum(a_ref, 0), wl_ref[...],
                preferred_element_type=jnp.float32)
    h = m * inv + jnp.dot(x_ref[...], wr_ref[...],
                          preferred_element_type=jnp.float32) + b_ref[...]
    h = jnp.maximum(h, 0.0)
    lo_ref[...] = h[:, :128]
    hi_ref[...] = h[:, 128:]


def _k4_body(a_ref, deg_ref, hlo_ref, hhi_ref, wl2_ref, wr2_ref, b2_ref,
             wl3_ref, wr3_ref, b3_ref, t3_ref, s3_ref):
    inv = _inv_deg(deg_ref)
    m = (jnp.dot(_psum(a_ref, 0), wl2_ref[0:128],
                 preferred_element_type=jnp.float32)
         + jnp.dot(_psum(a_ref, 1), wl2_ref[128:256],
                   preferred_element_type=jnp.float32))
    r = (jnp.dot(hlo_ref[...], wr2_ref[0:128],
                 preferred_element_type=jnp.float32)
         + jnp.dot(hhi_ref[...], wr2_ref[128:256],
                   preferred_element_type=jnp.float32))
    h2 = jnp.maximum(m * inv + r + b2_ref[...], 0.0)
    t3_ref[...] = jnp.dot(h2, wl3_ref[...], preferred_element_type=jnp.float32)
    s3_ref[...] = jnp.dot(h2, wr3_ref[...],
                          preferred_element_type=jnp.float32) + b3_ref[...]


def _k6_body(a_ref, deg_ref, s3_ref, out_ref):
    inv = _inv_deg(deg_ref)
    out_ref[...] = _psum(a_ref, 0) * inv + s3_ref[...]


@functools.lru_cache(maxsize=None)
def _make_k2(n, npad, bn=400):
    return pl.pallas_call(
        _k2_body,
        grid=(n // bn,),
        in_specs=[
            pl.BlockSpec((NC, bn, 128), lambda i: (0, i, 0)),
            pl.BlockSpec((NC, bn, 128), lambda i: (0, i, 0)),
            pl.BlockSpec((bn, 128), lambda i: (i, 0)),
            pl.BlockSpec((128, 256), lambda i: (0, 0)),
            pl.BlockSpec((128, 256), lambda i: (0, 0)),
            pl.BlockSpec((1, 256), lambda i: (0, 0)),
        ],
        out_specs=[
            pl.BlockSpec((bn, 128), lambda i: (i, 0)),
            pl.BlockSpec((bn, 128), lambda i: (i, 0)),
        ],
        out_shape=[
            jax.ShapeDtypeStruct((n, 128), jnp.float32),
            jax.ShapeDtypeStruct((n, 128), jnp.float32),
        ],
    )


@functools.lru_cache(maxsize=None)
def _make_k4(n, npad, bn=400):
    return pl.pallas_call(
        _k4_body,
        grid=(n // bn,),
        in_specs=[
            pl.BlockSpec((2 * NC, bn, 128), lambda i: (0, i, 0)),
            pl.BlockSpec((NC, bn, 128), lambda i: (0, i, 0)),
            pl.BlockSpec((bn, 128), lambda i: (i, 0)),
            pl.BlockSpec((bn, 128), lambda i: (i, 0)),
            pl.BlockSpec((256, 256), lambda i: (0, 0)),
            pl.BlockSpec((256, 256), lambda i: (0, 0)),
            pl.BlockSpec((1, 256), lambda i: (0, 0)),
            pl.BlockSpec((256, 128), lambda i: (0, 0)),
            pl.BlockSpec((256, 128), lambda i: (0, 0)),
            pl.BlockSpec((1, 128), lambda i: (0, 0)),
        ],
        out_specs=[
            pl.BlockSpec((bn, 128), lambda i: (i, 0)),
            pl.BlockSpec((bn, 128), lambda i: (i, 0)),
        ],
        out_shape=[
            jax.ShapeDtypeStruct((n, 128), jnp.float32),
            jax.ShapeDtypeStruct((n, 128), jnp.float32),
        ],
    )


@functools.lru_cache(maxsize=None)
def _make_k6(n, npad, bn=400):
    return pl.pallas_call(
        _k6_body,
        grid=(n // bn,),
        in_specs=[
            pl.BlockSpec((NC, bn, 128), lambda i: (0, i, 0)),
            pl.BlockSpec((NC, bn, 128), lambda i: (0, i, 0)),
            pl.BlockSpec((bn, 128), lambda i: (i, 0)),
        ],
        out_specs=pl.BlockSpec((bn, 128), lambda i: (i, 0)),
        out_shape=jax.ShapeDtypeStruct((n, 128), jnp.float32),
    )


@jax.jit
def kernel(x, edge_index, W_l1, W_r1, b1, W_l2, W_r2, b2, W_l3, W_r3, b3):
    n, d_in = x.shape
    e = edge_index.shape[1]

    npad = -(-n // 128) * 128

    trows = -(-e // (NS * CW * SG)) * SG
    rpt0 = min(trows - SG, max(SG, int(round(trows * 0.775 / SG)) * SG))
    rpt1 = trows - rpt0
    epad = NS * CW * trows
    src = jnp.concatenate(
        [edge_index[0], jnp.zeros((epad - e,), jnp.int32)])
    dst = jnp.concatenate(
        [edge_index[1], jnp.full((epad - e,), npad - 1, jnp.int32)])
    srcm = src.reshape(epad // CW, CW)
    dstm = dst.reshape(epad // CW, CW)

    agg1, deg = _make_sc_segsum(npad, rpt0, rpt1, 1, True)(x, srcm, dstm)
    h1lo, h1hi = _make_k2(n, npad)(agg1, deg, x, W_l1, W_r1, b1.reshape(1, -1))
    agg2 = _make_sc_segsum(npad, rpt0, rpt1, 2)(h1lo, h1hi, srcm, dstm)
    t3, s3 = _make_k4(n, npad)(agg2, deg, h1lo, h1hi, W_l2, W_r2,
                               b2.reshape(1, -1), W_l3, W_r3,
                               b3.reshape(1, -1))
    agg3 = _make_sc_segsum(npad, rpt0, rpt1, 1)(t3, srcm, dstm)
    return _make_k6(n, npad)(agg3, deg, s3)
